# baseline (speedup 1.0000x reference)
import functools

import jax
import jax.numpy as jnp
from jax import lax
from jax.experimental import pallas as pl
from jax.experimental.pallas import tpu as pltpu
from jax.experimental.pallas import tpu_sc as plsc

D_MODEL = 128
NUM_CORES = 2
NUM_SUBCORES = 16
NUM_WORKERS = NUM_CORES * NUM_SUBCORES

ROWS_PER_STREAM = 128
STREAMS_PER_GROUP = 2


def _make_gather(M: int, V: int):
  assert M % (NUM_WORKERS * STREAMS_PER_GROUP) == 0
  m_per_w = M // NUM_WORKERS
  n_groups = m_per_w // STREAMS_PER_GROUP
  assert n_groups % 2 == 0

  mesh = plsc.VectorSubcoreMesh(
      core_axis_name="c", subcore_axis_name="s",
      num_cores=NUM_CORES, num_subcores=NUM_SUBCORES)

  @functools.partial(
      pl.kernel,
      out_type=jax.ShapeDtypeStruct((M, ROWS_PER_STREAM, D_MODEL),
                                    jnp.float32),
      mesh=mesh,
      scratch_types=[
          pltpu.VMEM((2, STREAMS_PER_GROUP, ROWS_PER_STREAM), jnp.int32),
          pltpu.VMEM((2, STREAMS_PER_GROUP, ROWS_PER_STREAM, D_MODEL),
                     jnp.float32),
          pltpu.VMEM_SHARED((V, D_MODEL), jnp.float32),
          pltpu.SemaphoreType.DMA,
          pltpu.SemaphoreType.DMA,
          pltpu.SemaphoreType.DMA,
          pltpu.SemaphoreType.DMA,
      ],
  )
  def gather_kernel(idx_hbm, table_hbm, out_hbm, idx_v, rows_v, table_sp,
                    gsem0, gsem1, ssem0, ssem1):
    wid = lax.axis_index("s") * NUM_CORES + lax.axis_index("c")
    m_base = wid * m_per_w
    gsem = (gsem0, gsem1)
    ssem = (ssem0, ssem1)

    @pl.when(lax.axis_index("s") == 0)
    def _():
      pltpu.sync_copy(table_hbm, table_sp)

    plsc.subcore_barrier()

    def fire_gather(g, buf):
      m0 = m_base + g * STREAMS_PER_GROUP
      pltpu.sync_copy(idx_hbm.at[pl.ds(m0, STREAMS_PER_GROUP)],
                      idx_v.at[buf])
      for j in range(STREAMS_PER_GROUP):
        pltpu.async_copy(
            table_sp.at[idx_v.at[buf].at[j]],
            rows_v.at[buf].at[j],
            gsem[buf])

    def wait_gather(buf):
      for j in range(STREAMS_PER_GROUP):
        pltpu.make_async_copy(
            table_sp.at[idx_v.at[buf].at[j]],
            rows_v.at[buf].at[j],
            gsem[buf]).wait()

    def wait_scatter(buf):
      pltpu.make_async_copy(
          rows_v.at[buf], out_hbm.at[pl.ds(m_base, STREAMS_PER_GROUP)],
          ssem[buf]).wait()

    fire_gather(0, 0)

    @pl.loop(0, n_groups // 2)
    def _(p):
      for buf in (0, 1):
        g = 2 * p + buf
        other = 1 - buf

        @pl.when(g + 1 < n_groups)
        def _():
          @pl.when(g >= 1)
          def _():
            wait_scatter(other)
          fire_gather(g + 1, other)

        wait_gather(buf)
        pltpu.async_copy(
            rows_v.at[buf],
            out_hbm.at[pl.ds(m_base + g * STREAMS_PER_GROUP,
                             STREAMS_PER_GROUP)],
            ssem[buf])

    wait_scatter(0)
    wait_scatter(1)

  return gather_kernel


def kernel(var_indices, var_embedding):
  n, s = var_indices.shape
  d = var_embedding.shape[1]
  m = (n * s) // ROWS_PER_STREAM
  idx = var_indices.astype(jnp.int32).reshape(m, ROWS_PER_STREAM)
  out = _make_gather(m, var_embedding.shape[0])(idx, var_embedding)
  return out.reshape(n, s, d)


# speedup vs baseline: 2.0641x; 2.0641x over previous
import functools

import jax
import jax.numpy as jnp
from jax import lax
from jax.experimental import pallas as pl
from jax.experimental.pallas import tpu as pltpu
from jax.experimental.pallas import tpu_sc as plsc

D_MODEL = 128
NUM_CORES = 2
NUM_SUBCORES = 16
NUM_WORKERS = NUM_CORES * NUM_SUBCORES

TOKENS_PER_GROUP = 8


def _make_gather(N: int, S: int, V: int):
  assert N % (NUM_WORKERS * TOKENS_PER_GROUP) == 0
  t_per_w = N // NUM_WORKERS
  n_groups = t_per_w // TOKENS_PER_GROUP
  assert n_groups % 2 == 0

  mesh = plsc.VectorSubcoreMesh(
      core_axis_name="c", subcore_axis_name="s",
      num_cores=NUM_CORES, num_subcores=NUM_SUBCORES)

  @functools.partial(
      pl.kernel,
      out_type=jax.ShapeDtypeStruct((N, S, D_MODEL), jnp.float32),
      mesh=mesh,
      scratch_types=[
          pltpu.VMEM((2, TOKENS_PER_GROUP, S), jnp.int32),
          pltpu.VMEM((2, TOKENS_PER_GROUP, S, D_MODEL), jnp.float32),
          pltpu.VMEM_SHARED((V, D_MODEL), jnp.float32),
          pltpu.SemaphoreType.DMA,
          pltpu.SemaphoreType.DMA,
          pltpu.SemaphoreType.DMA,
          pltpu.SemaphoreType.DMA,
      ],
  )
  def gather_kernel(idx_hbm, table_hbm, out_hbm, idx_v, rows_v, table_sp,
                    gsem0, gsem1, ssem0, ssem1):
    wid = lax.axis_index("s") * NUM_CORES + lax.axis_index("c")
    tok_base = wid * t_per_w
    gsem = (gsem0, gsem1)
    ssem = (ssem0, ssem1)

    @pl.when(lax.axis_index("s") == 0)
    def _():
      pltpu.sync_copy(table_hbm, table_sp)

    plsc.subcore_barrier()

    def fire_gather(g, buf):
      tok0 = tok_base + g * TOKENS_PER_GROUP
      pltpu.sync_copy(idx_hbm.at[pl.ds(tok0, TOKENS_PER_GROUP)],
                      idx_v.at[buf])
      for j in range(TOKENS_PER_GROUP):
        pltpu.async_copy(
            table_sp.at[idx_v.at[buf].at[j]],
            rows_v.at[buf].at[j],
            gsem[buf])

    def wait_gather(buf):
      for j in range(TOKENS_PER_GROUP):
        pltpu.make_async_copy(
            table_sp.at[idx_v.at[buf].at[j]],
            rows_v.at[buf].at[j],
            gsem[buf]).wait()

    def wait_scatter(buf):
      pltpu.make_async_copy(
          rows_v.at[buf], out_hbm.at[pl.ds(tok_base, TOKENS_PER_GROUP)],
          ssem[buf]).wait()

    fire_gather(0, 0)

    @pl.loop(0, n_groups // 2)
    def _(p):
      for buf in (0, 1):
        g = 2 * p + buf
        other = 1 - buf

        @pl.when(g + 1 < n_groups)
        def _():
          @pl.when(g >= 1)
          def _():
            wait_scatter(other)
          fire_gather(g + 1, other)

        wait_gather(buf)
        pltpu.async_copy(
            rows_v.at[buf],
            out_hbm.at[pl.ds(tok_base + g * TOKENS_PER_GROUP,
                             TOKENS_PER_GROUP)],
            ssem[buf])

    wait_scatter(0)
    wait_scatter(1)

  return gather_kernel


def kernel(var_indices, var_embedding):
  n, s = var_indices.shape
  idx = var_indices.astype(jnp.int32)
  return _make_gather(n, s, var_embedding.shape[0])(idx, var_embedding)
